# baseline (speedup 1.0000x reference)
import jax
import jax.numpy as jnp
from jax.experimental import pallas as pl
from jax.experimental.pallas import tpu as pltpu

C = 8
K = 8192
D = 256
S = 576
B = 16
KW = 2736
KB = 1024


def _vq_kernel(xt_ref, cb_ref, zq_ref, idx_ref, xres_ref, zqacc_ref):
    c = pl.program_id(0)
    b = pl.program_id(1)
    rows = pl.ds(b * S, S)

    @pl.when(c == 0)
    def _init():
        xres_ref[rows, :] = xt_ref[...]
        zqacc_ref[rows, :] = jnp.zeros((S, D), jnp.float32)

    x = xres_ref[rows, :]
    xh = x.astype(jnp.bfloat16)
    xx = jax.lax.dot_general(
        jnp.ones((1, D), jnp.float32), x * x, (((1,), (1,)), ((), ())),
        preferred_element_type=jnp.float32,
        precision=jax.lax.Precision.HIGHEST)

    def dg(a, b_, dims):
        return jax.lax.dot_general(a, b_, dims,
                                   preferred_element_type=jnp.float32)

    dims_nt = (((1,), (1,)), ((), ()))
    best_val = jnp.full((1, S), jnp.inf, jnp.float32)
    best_idx = jnp.zeros((1, S), jnp.int32)
    is_last = c == C - 1
    for k0 in range(0, K, KW):
        kw = min(KW, K - k0)
        e = cb_ref[0, pl.ds(k0, kw), :]
        ee = jnp.sum(e * e, axis=1, keepdims=True)
        prod = dg(e.astype(jnp.bfloat16), xh, dims_nt)
        v = (xx - 2.0 * prod) + ee
        m = jnp.min(v, axis=0, keepdims=True)
        ii = jax.lax.broadcasted_iota(jnp.int32, (kw, S), 0) + k0
        idxc = jnp.min(jnp.where(v == m, ii, jnp.int32(K)),
                       axis=0, keepdims=True)
        upd = m < best_val
        best_val = jnp.where(upd, m, best_val)
        best_idx = jnp.where(upd, idxc, best_idx)
        qv = best_val.astype(jnp.bfloat16).astype(jnp.float32)
        best_val = jnp.where(is_last, qv, best_val)

    def _gather_step(i, z):
        k0 = pl.multiple_of(i * KB, KB)
        e = cb_ref[0, pl.ds(k0, KB), :]
        e1 = e.astype(jnp.bfloat16)
        r1 = e - e1.astype(jnp.float32)
        e2 = r1.astype(jnp.bfloat16)
        e3 = (r1 - e2.astype(jnp.float32)).astype(jnp.bfloat16)
        ii = jax.lax.broadcasted_iota(jnp.int32, (KB, S), 0) + k0
        oh = (ii == best_idx).astype(jnp.bfloat16)
        dims = (((0,), (0,)), ((), ()))
        return z + (dg(oh, e1, dims) + dg(oh, e2, dims) + dg(oh, e3, dims))

    z = jax.lax.fori_loop(0, K // KB, _gather_step,
                          jnp.zeros((S, D), jnp.float32))

    xres_ref[rows, :] = x - z
    zq = zqacc_ref[rows, :] + z
    zqacc_ref[rows, :] = zq
    zq_ref[0, 0, :, :] = zq
    idx_ref[0, :, :] = best_idx


def kernel(x_in, codebooks):
    Bb, Dd, Ss = x_in.shape
    xt = jnp.transpose(x_in, (0, 2, 1)).reshape(Bb * Ss, Dd)

    zq_out, idx_out = pl.pallas_call(
        _vq_kernel,
        grid=(C, B),
        in_specs=[
            pl.BlockSpec((S, D), lambda c, b: (b, 0)),
            pl.BlockSpec((1, K, D), lambda c, b: (c, 0, 0)),
        ],
        out_specs=[
            pl.BlockSpec((1, 1, S, D), lambda c, b: (c, b, 0, 0)),
            pl.BlockSpec((1, 1, S), lambda c, b: (c * B + b, 0, 0)),
        ],
        out_shape=[
            jax.ShapeDtypeStruct((C, B, S, D), jnp.float32),
            jax.ShapeDtypeStruct((C * B, 1, S), jnp.int32),
        ],
        scratch_shapes=[
            pltpu.VMEM((Bb * Ss, Dd), jnp.float32),
            pltpu.VMEM((Bb * Ss, Dd), jnp.float32),
        ],
    )(xt, codebooks)

    z_q_aggregated = jnp.transpose(zq_out, (1, 0, 3, 2))
    indices = jnp.transpose(idx_out.reshape(C, Bb, Ss), (1, 2, 0))
    return z_q_aggregated, indices


# speedup vs baseline: 1.0011x; 1.0011x over previous
import jax
import jax.numpy as jnp
from jax.experimental import pallas as pl
from jax.experimental.pallas import tpu as pltpu

C = 8
K = 8192
D = 256
S = 576
B = 16
KW = 2736
KB = 1024


def _vq_kernel(xt_ref, cb_ref, zq_ref, idx_ref, xres_ref, zqacc_ref):
    c = pl.program_id(0)
    b = pl.program_id(1)
    rows = pl.ds(b * S, S)

    @pl.when(c == 0)
    def _init():
        xres_ref[rows, :] = xt_ref[...]
        zqacc_ref[rows, :] = jnp.zeros((S, D), jnp.float32)

    x = xres_ref[rows, :]
    xh = x.astype(jnp.bfloat16)
    xx = jax.lax.dot_general(
        jnp.ones((1, D), jnp.float32), x * x, (((1,), (1,)), ((), ())),
        preferred_element_type=jnp.float32,
        precision=jax.lax.Precision.HIGHEST)

    def dg(a, b_, dims):
        return jax.lax.dot_general(a, b_, dims,
                                   preferred_element_type=jnp.float32)

    dims_nt = (((1,), (1,)), ((), ()))
    best_val = jnp.full((1, S), jnp.inf, jnp.float32)
    best_idx = jnp.zeros((1, S), jnp.int32)
    is_last = c == C - 1
    for k0 in range(0, K, KW):
        kw = min(KW, K - k0)
        e = cb_ref[0, pl.ds(k0, kw), :]
        ee = jnp.sum(e * e, axis=1, keepdims=True)
        prod = dg(e.astype(jnp.bfloat16), xh, dims_nt)
        v = (xx - 2.0 * prod) + ee
        m = jnp.min(v, axis=0, keepdims=True)
        ii = jax.lax.broadcasted_iota(jnp.int32, (kw, S), 0) + k0
        idxc = jnp.min(jnp.where(v == m, ii, jnp.int32(K)),
                       axis=0, keepdims=True)
        upd = m < best_val
        best_val = jnp.where(upd, m, best_val)
        best_idx = jnp.where(upd, idxc, best_idx)
        qv = best_val.astype(jnp.bfloat16).astype(jnp.float32)
        best_val = jnp.where(is_last, qv, best_val)

    def _gather_step(i, z):
        k0 = pl.multiple_of(i * KB, KB)
        e = cb_ref[0, pl.ds(k0, KB), :]
        e1 = e.astype(jnp.bfloat16)
        r1 = e - e1.astype(jnp.float32)
        e2 = r1.astype(jnp.bfloat16)
        e3 = (r1 - e2.astype(jnp.float32)).astype(jnp.bfloat16)
        ii = jax.lax.broadcasted_iota(jnp.int32, (KB, S), 0) + k0
        oh = (ii == best_idx).astype(jnp.bfloat16)
        dims = (((0,), (0,)), ((), ()))
        return z + (dg(oh, e1, dims) + dg(oh, e2, dims) + dg(oh, e3, dims))

    z = jax.lax.fori_loop(0, K // KB, _gather_step,
                          jnp.zeros((S, D), jnp.float32))

    xres_ref[rows, :] = x - z
    zq = zqacc_ref[rows, :] + z
    zqacc_ref[rows, :] = zq
    zq_ref[0, 0, :, :] = zq
    idx_ref[0, :, :] = best_idx


def kernel(x_in, codebooks):
    Bb, Dd, Ss = x_in.shape
    xt = jnp.transpose(x_in, (0, 2, 1)).reshape(Bb * Ss, Dd)

    zq_out, idx_out = pl.pallas_call(
        _vq_kernel,
        grid=(C, B),
        in_specs=[
            pl.BlockSpec((S, D), lambda c, b: (b, 0)),
            pl.BlockSpec((1, K, D), lambda c, b: (c, 0, 0)),
        ],
        out_specs=[
            pl.BlockSpec((1, 1, S, D), lambda c, b: (c, b, 0, 0)),
            pl.BlockSpec((1, 1, S), lambda c, b: (c * B + b, 0, 0)),
        ],
        out_shape=[
            jax.ShapeDtypeStruct((C, B, S, D), jnp.float32),
            jax.ShapeDtypeStruct((C * B, 1, S), jnp.int32),
        ],
        scratch_shapes=[
            pltpu.VMEM((Bb * Ss, Dd), jnp.float32),
            pltpu.VMEM((Bb * Ss, Dd), jnp.float32),
        ],
        compiler_params=pltpu.CompilerParams(
            dimension_semantics=("arbitrary", "parallel")),
    )(xt, codebooks)

    z_q_aggregated = jnp.transpose(zq_out, (1, 0, 3, 2))
    indices = jnp.transpose(idx_out.reshape(C, Bb, Ss), (1, 2, 0))
    return z_q_aggregated, indices


# speedup vs baseline: 1.0132x; 1.0122x over previous
import jax
import jax.numpy as jnp
from jax.experimental import pallas as pl
from jax.experimental.pallas import tpu as pltpu

C = 8
K = 8192
D = 256
S = 576
B = 16
KW = 2736
KB = 1024


def _vq_kernel(xt_ref, cb_ref, zq_ref, idx_ref, xres_ref, zqacc_ref,
               e1_ref, e2_ref):
    c = pl.program_id(0)
    b = pl.program_id(1)
    rows = pl.ds(b * S, S)

    @pl.when(c == 0)
    def _init():
        xres_ref[rows, :] = xt_ref[...]
        zqacc_ref[rows, :] = jnp.zeros((S, D), jnp.float32)

    @pl.when(b == 0)
    def _prep_codebook():
        for j in range(0, K, KB):
            ch = cb_ref[0, pl.ds(j, KB), :]
            e1 = ch.astype(jnp.bfloat16)
            e2 = (ch - e1.astype(jnp.float32)).astype(jnp.bfloat16)
            sl = pl.ds(j, KB)
            e1_ref[sl, :] = e1
            e2_ref[sl, :] = e2

    x = xres_ref[rows, :]
    xh = x.astype(jnp.bfloat16)
    xx = jax.lax.dot_general(
        jnp.ones((1, D), jnp.float32), x * x, (((1,), (1,)), ((), ())),
        preferred_element_type=jnp.float32,
        precision=jax.lax.Precision.HIGHEST)

    def dg(a, b_, dims):
        return jax.lax.dot_general(a, b_, dims,
                                   preferred_element_type=jnp.float32)

    dims_nt = (((1,), (1,)), ((), ()))
    best_val = jnp.full((1, S), jnp.inf, jnp.float32)
    best_idx = jnp.zeros((1, S), jnp.int32)
    is_last = c == C - 1
    for k0 in range(0, K, KW):
        kw = min(KW, K - k0)
        e = cb_ref[0, pl.ds(k0, kw), :]
        ee = jnp.sum(e * e, axis=1, keepdims=True)
        prod = dg(e1_ref[pl.ds(k0, kw), :], xh, dims_nt)
        v = (xx - 2.0 * prod) + ee
        m = jnp.min(v, axis=0, keepdims=True)
        ii = jax.lax.broadcasted_iota(jnp.int32, (kw, S), 0) + k0
        idxc = jnp.min(jnp.where(v == m, ii, jnp.int32(K)),
                       axis=0, keepdims=True)
        upd = m < best_val
        best_val = jnp.where(upd, m, best_val)
        best_idx = jnp.where(upd, idxc, best_idx)
        qv = best_val.astype(jnp.bfloat16).astype(jnp.float32)
        best_val = jnp.where(is_last, qv, best_val)

    def _gather_step(i, z):
        k0 = pl.multiple_of(i * KB, KB)
        sl = pl.ds(k0, KB)
        e1 = e1_ref[sl, :]
        e2 = e2_ref[sl, :]
        e3 = ((cb_ref[0, sl, :] - e1.astype(jnp.float32))
              - e2.astype(jnp.float32)).astype(jnp.bfloat16)
        ii = jax.lax.broadcasted_iota(jnp.int32, (KB, S), 0) + k0
        oh = (ii == best_idx).astype(jnp.bfloat16)
        dims = (((0,), (0,)), ((), ()))
        return z + (dg(oh, e1, dims) + dg(oh, e2, dims) + dg(oh, e3, dims))

    z = jax.lax.fori_loop(0, K // KB, _gather_step,
                          jnp.zeros((S, D), jnp.float32))

    xres_ref[rows, :] = x - z
    zq = zqacc_ref[rows, :] + z
    zqacc_ref[rows, :] = zq
    zq_ref[0, 0, :, :] = zq
    idx_ref[0, :, :] = best_idx


def kernel(x_in, codebooks):
    Bb, Dd, Ss = x_in.shape
    xt = jnp.transpose(x_in, (0, 2, 1)).reshape(Bb * Ss, Dd)

    zq_out, idx_out = pl.pallas_call(
        _vq_kernel,
        grid=(C, B),
        in_specs=[
            pl.BlockSpec((S, D), lambda c, b: (b, 0)),
            pl.BlockSpec((1, K, D), lambda c, b: (c, 0, 0)),
        ],
        out_specs=[
            pl.BlockSpec((1, 1, S, D), lambda c, b: (c, b, 0, 0)),
            pl.BlockSpec((1, 1, S), lambda c, b: (c * B + b, 0, 0)),
        ],
        out_shape=[
            jax.ShapeDtypeStruct((C, B, S, D), jnp.float32),
            jax.ShapeDtypeStruct((C * B, 1, S), jnp.int32),
        ],
        scratch_shapes=[
            pltpu.VMEM((Bb * Ss, Dd), jnp.float32),
            pltpu.VMEM((Bb * Ss, Dd), jnp.float32),
            pltpu.VMEM((K, D), jnp.bfloat16),
            pltpu.VMEM((K, D), jnp.bfloat16),
        ],
        compiler_params=pltpu.CompilerParams(
            dimension_semantics=("arbitrary", "parallel")),
    )(xt, codebooks)

    z_q_aggregated = jnp.transpose(zq_out, (1, 0, 3, 2))
    indices = jnp.transpose(idx_out.reshape(C, Bb, Ss), (1, 2, 0))
    return z_q_aggregated, indices
